# speedup vs baseline: 4.4594x; 4.4594x over previous
import functools

import jax
import jax.numpy as jnp
from jax import lax
from jax.experimental import pallas as pl
from jax.experimental.pallas import tpu as pltpu
from jax.experimental.pallas import tpu_sc as plsc

N = 10000
E = 320000
D = 128

NC = 2
NS = 16
NW = NC * NS
CHUNK = 128
EPW = 10112
E_PAD = NW * EPW
N_PAD = 10240
RPS = N_PAD // NS
DUMMY = N


def _sc_spmm(hmu, srcp, dstp):
    mesh = plsc.VectorSubcoreMesh(core_axis_name="c", subcore_axis_name="s")
    part = jax.ShapeDtypeStruct((N_PAD, D), jnp.float32)

    @functools.partial(
        pl.kernel,
        out_type=[part, part],
        mesh=mesh,
        scratch_types=[
            pltpu.VMEM((CHUNK,), jnp.int32),
            pltpu.VMEM((CHUNK,), jnp.int32),
            pltpu.VMEM((CHUNK, D), jnp.float32),
            pltpu.VMEM_SHARED((N_PAD, D), jnp.float32),
            pltpu.SemaphoreType.DMA,
        ],
    )
    def spmm(hmu_hbm, src_hbm, dst_hbm, out0_hbm, out1_hbm,
             src_v, dst_v, rows_v, acc, sem):
        cid = lax.axis_index("c")
        sid = lax.axis_index("s")
        row0 = sid * RPS

        @pl.loop(0, CHUNK)
        def _(i):
            @pl.loop(0, D, step=16)
            def _(j):
                rows_v[i, pl.ds(j, 16)] = jnp.zeros((16,), jnp.float32)

        @pl.loop(0, RPS, step=CHUNK)
        def _(r):
            pltpu.sync_copy(rows_v, acc.at[pl.ds(row0 + r, CHUNK)])

        plsc.subcore_barrier()

        base = (sid * NC + cid) * EPW

        @pl.loop(0, EPW, step=CHUNK)
        def _(e0):
            off = base + e0
            pltpu.sync_copy(src_hbm.at[pl.ds(off, CHUNK)], src_v)
            pltpu.sync_copy(dst_hbm.at[pl.ds(off, CHUNK)], dst_v)
            pltpu.async_copy(hmu_hbm.at[src_v], rows_v, sem).wait()
            pltpu.sync_copy(rows_v, acc.at[dst_v], add=True)

        plsc.subcore_barrier()

        @pl.when(cid == 0)
        def _():
            pltpu.sync_copy(acc.at[pl.ds(row0, RPS)],
                            out0_hbm.at[pl.ds(row0, RPS)])

        @pl.when(cid == 1)
        def _():
            pltpu.sync_copy(acc.at[pl.ds(row0, RPS)],
                            out1_hbm.at[pl.ds(row0, RPS)])

    return spmm(hmu, srcp, dstp)


def _tc1_body(x_ref, wpt_ref, bp_ref, w1t_ref, b12_ref, hmu_ref, y1_ref):
    h = jnp.dot(x_ref[...], wpt_ref[...], preferred_element_type=jnp.float32)
    h = jnp.maximum(h + bp_ref[...], 0.0)
    hmu_ref[...] = h * h
    y1_ref[...] = (jnp.dot(h, w1t_ref[...], preferred_element_type=jnp.float32)
                   + b12_ref[...])


def _tc2_body(y1_ref, a0_ref, a1_ref, w2t_ref, out_ref):
    agg = jnp.sqrt(a0_ref[...] + a1_ref[...])
    out_ref[...] = y1_ref[...] + jnp.dot(
        agg, w2t_ref[...], preferred_element_type=jnp.float32)


_BLK = 1000


def kernel(x, edge_index, adj_values, W_pool, b_pool, W1, b1, W2, b2):
    del adj_values
    src = edge_index[1]
    dst = edge_index[0]
    pad = E_PAD - E
    srcp = jnp.concatenate([src, jnp.zeros((pad,), jnp.int32)])
    dstp = jnp.concatenate([dst, jnp.full((pad,), DUMMY, jnp.int32)])

    row_spec = pl.BlockSpec((_BLK, D), lambda i: (i, 0))
    mat_spec = pl.BlockSpec((D, D), lambda i: (0, 0))
    bias_spec = pl.BlockSpec((1, D), lambda i: (0, 0))

    hmu, y1 = pl.pallas_call(
        _tc1_body,
        grid=(N // _BLK,),
        in_specs=[row_spec, mat_spec, bias_spec, mat_spec, bias_spec],
        out_specs=[row_spec, row_spec],
        out_shape=[jax.ShapeDtypeStruct((N, D), jnp.float32),
                   jax.ShapeDtypeStruct((N, D), jnp.float32)],
    )(x, W_pool.T, b_pool[None, :], W1.T, (b1 + b2)[None, :])

    p0, p1 = _sc_spmm(hmu, srcp, dstp)

    out = pl.pallas_call(
        _tc2_body,
        grid=(N // _BLK,),
        in_specs=[row_spec, row_spec, row_spec, mat_spec],
        out_specs=row_spec,
        out_shape=jax.ShapeDtypeStruct((N, D), jnp.float32),
    )(y1, p0, p1, W2.T)
    return out


# speedup vs baseline: 4.5740x; 1.0257x over previous
import functools

import jax
import jax.numpy as jnp
from jax import lax
from jax.experimental import pallas as pl
from jax.experimental.pallas import tpu as pltpu
from jax.experimental.pallas import tpu_sc as plsc

N = 10000
E = 320000
D = 128

NC = 2
NS = 16
CHUNK = 128
NB = 2
PCH = 32
PH0 = 4
PH1 = 1
PH = PH0 + PH1
E_PAD = NS * PH * PCH * CHUNK
N_PAD = 10240
RPS = N_PAD // NS
DUMMY = N


def _sc_spmm(hmu, srcp, dstp):
    mesh = plsc.VectorSubcoreMesh(core_axis_name="c", subcore_axis_name="s")
    part = jax.ShapeDtypeStruct((N_PAD, D), jnp.float32)

    @functools.partial(
        pl.kernel,
        out_type=[part, part],
        mesh=mesh,
        scratch_types=[
            pltpu.VMEM((PCH, CHUNK), jnp.int32),
            pltpu.VMEM((PCH, CHUNK), jnp.int32),
            pltpu.VMEM((NB, CHUNK, D), jnp.float32),
            pltpu.VMEM_SHARED((N_PAD, D), jnp.float32),
            pltpu.SemaphoreType.DMA,
            pltpu.SemaphoreType.DMA,
            pltpu.SemaphoreType.DMA,
        ],
    )
    def spmm(hmu_hbm, src_hbm, dst_hbm, out0_hbm, out1_hbm,
             src_all, dst_all, rows, acc, isem, g0, g1):
        gsems = (g0, g1)
        cid = lax.axis_index("c")
        sid = lax.axis_index("s")
        row0 = sid * RPS

        @pl.loop(0, CHUNK)
        def _(i):
            @pl.loop(0, D, step=16)
            def _(j):
                rows[0, i, pl.ds(j, 16)] = jnp.zeros((16,), jnp.float32)

        @pl.loop(0, RPS, step=CHUNK)
        def _(r):
            pltpu.sync_copy(rows.at[0], acc.at[pl.ds(row0 + r, CHUNK)])

        plsc.subcore_barrier()

        def _chunk(c, b):
            pltpu.make_async_copy(
                hmu_hbm.at[src_all.at[c]], rows.at[b], gsems[b]).wait()
            pltpu.sync_copy(rows.at[b], acc.at[dst_all.at[c]], add=True)

            @pl.when(c + NB < PCH)
            def _():
                pltpu.async_copy(
                    hmu_hbm.at[src_all.at[c + NB]], rows.at[b], gsems[b])

        def _run_block(p):
            pltpu.async_copy(src_hbm.at[sid, p], src_all, isem).wait()
            pltpu.async_copy(dst_hbm.at[sid, p], dst_all, isem).wait()
            for b in range(NB):
                pltpu.async_copy(
                    hmu_hbm.at[src_all.at[b]], rows.at[b], gsems[b])

            @pl.loop(0, PCH, step=NB)
            def _(k):
                for b in range(NB):
                    _chunk(k + b, b)

        @pl.when(cid == 0)
        def _():
            for p in range(PH0):
                _run_block(p)

        @pl.when(cid == 1)
        def _():
            for p in range(PH0, PH):
                _run_block(p)

        plsc.subcore_barrier()

        @pl.when(cid == 0)
        def _():
            pltpu.sync_copy(acc.at[pl.ds(row0, RPS)],
                            out0_hbm.at[pl.ds(row0, RPS)])

        @pl.when(cid == 1)
        def _():
            pltpu.sync_copy(acc.at[pl.ds(row0, RPS)],
                            out1_hbm.at[pl.ds(row0, RPS)])

    return spmm(hmu, srcp, dstp)


def _tc1_body(x_ref, wpt_ref, bp_ref, w1t_ref, b12_ref, hmu_ref, y1_ref):
    h = jnp.dot(x_ref[...], wpt_ref[...], preferred_element_type=jnp.float32)
    h = jnp.maximum(h + bp_ref[...], 0.0)
    hmu_ref[...] = h * h
    y1_ref[...] = (jnp.dot(h, w1t_ref[...], preferred_element_type=jnp.float32)
                   + b12_ref[...])


def _tc2_body(y1_ref, a0_ref, a1_ref, w2t_ref, out_ref):
    agg = jnp.sqrt(a0_ref[...] + a1_ref[...])
    out_ref[...] = y1_ref[...] + jnp.dot(
        agg, w2t_ref[...], preferred_element_type=jnp.float32)


_BLK = 1000


def kernel(x, edge_index, adj_values, W_pool, b_pool, W1, b1, W2, b2):
    del adj_values
    src = edge_index[1]
    dst = edge_index[0]
    pad = E_PAD - E
    srcp = jnp.concatenate([src, jnp.zeros((pad,), jnp.int32)])
    dstp = jnp.concatenate([dst, jnp.full((pad,), DUMMY, jnp.int32)])
    srcp = srcp.reshape(NS, PH, PCH, CHUNK)
    dstp = dstp.reshape(NS, PH, PCH, CHUNK)

    row_spec = pl.BlockSpec((_BLK, D), lambda i: (i, 0))
    mat_spec = pl.BlockSpec((D, D), lambda i: (0, 0))
    bias_spec = pl.BlockSpec((1, D), lambda i: (0, 0))

    hmu, y1 = pl.pallas_call(
        _tc1_body,
        grid=(N // _BLK,),
        in_specs=[row_spec, mat_spec, bias_spec, mat_spec, bias_spec],
        out_specs=[row_spec, row_spec],
        out_shape=[jax.ShapeDtypeStruct((N, D), jnp.float32),
                   jax.ShapeDtypeStruct((N, D), jnp.float32)],
    )(x, W_pool.T, b_pool[None, :], W1.T, (b1 + b2)[None, :])

    p0, p1 = _sc_spmm(hmu, srcp, dstp)

    out = pl.pallas_call(
        _tc2_body,
        grid=(N // _BLK,),
        in_specs=[row_spec, row_spec, row_spec, mat_spec],
        out_specs=row_spec,
        out_shape=jax.ShapeDtypeStruct((N, D), jnp.float32),
    )(y1, p0, p1, W2.T)
    return out
